# baseline (speedup 1.0000x reference)
import jax
import jax.numpy as jnp
from jax.experimental import pallas as pl
from jax.experimental.pallas import tpu as pltpu

N = 10000
E = 320000
IN_DIM = 128
OUT_DIM = 64


def _combine_body(xg_ref, xa_ref, wg_ref, wa_ref, out_ref):
    xg = xg_ref[...]
    xa = xa_ref[...]
    y = jnp.dot(xg, wg_ref[...], preferred_element_type=jnp.float32,
                precision=jax.lax.Precision.HIGHEST)
    y += jnp.dot(xa, wa_ref[...], preferred_element_type=jnp.float32,
                 precision=jax.lax.Precision.HIGHEST)
    out_ref[...] = jnp.tanh(y)


def _combine(xg, xa, w_gcn, w_gat):
    return pl.pallas_call(
        _combine_body,
        out_shape=jax.ShapeDtypeStruct((N, OUT_DIM), jnp.float32),
    )(xg, xa, w_gcn, w_gat)


def kernel(x, edge_index, W_gcn, b_gcn, W_gat, att_src, att_dst, b_gat, w_gcn, w_gat):
    n = x.shape[0]
    loop = jnp.arange(n, dtype=edge_index.dtype)
    src = jnp.concatenate([edge_index[0], loop])
    dst = jnp.concatenate([edge_index[1], loop])

    deg = jax.ops.segment_sum(jnp.ones(src.shape[0], jnp.float32), dst, num_segments=n)
    dinv = jnp.where(deg > 0, deg ** -0.5, 0.0)
    norm = dinv[src] * dinv[dst]
    h = x @ W_gcn
    x_gcn = jax.ops.segment_sum(h[src] * norm[:, None], dst, num_segments=n) + b_gcn

    h2 = x @ W_gat
    alpha_src = h2 @ att_src
    alpha_dst = h2 @ att_dst
    e = jax.nn.leaky_relu(alpha_src[src] + alpha_dst[dst], negative_slope=0.2)
    emax = jax.ops.segment_max(e, dst, num_segments=n)
    ee = jnp.exp(e - emax[dst])
    denom = jax.ops.segment_sum(ee, dst, num_segments=n)
    alpha = ee / denom[dst]
    x_gat = jax.ops.segment_sum(h2[src] * alpha[:, None], dst, num_segments=n) + b_gat

    return _combine(x_gcn, x_gat, w_gcn, w_gat)


# speedup vs baseline: 22.9561x; 22.9561x over previous
import functools

import jax
import jax.numpy as jnp
from jax import lax
from jax.experimental import pallas as pl
from jax.experimental.pallas import tpu as pltpu
from jax.experimental.pallas import tpu_sc as plsc

N = 10000
E = 320000
IN_DIM = 128
OUT_DIM = 64

NC = 2
NS = 16
NW = NC * NS
NPAD = 10240
CH = 128
NCHUNK = 80
EPW = NCHUNK * CH
EPAD = NW * EPW
NROWS_T = NPAD // NS

_HIGH = lax.Precision.HIGHEST

_mesh = plsc.VectorSubcoreMesh(
    core_axis_name="c", subcore_axis_name="s", num_cores=NC, num_subcores=NS
)
_sc_params = pltpu.CompilerParams(needs_layout_passes=False)


def _prep_body(x_ref, wgcn_ref, wg_ref, wgat_ref, wa_ref, att2_ref,
               g_ref, asad_ref):
    x = x_ref[...]
    m1 = jnp.dot(wgcn_ref[...], wg_ref[...],
                 preferred_element_type=jnp.float32, precision=_HIGH)
    m2 = jnp.dot(wgat_ref[...], wa_ref[...],
                 preferred_element_type=jnp.float32, precision=_HIGH)
    g1 = jnp.dot(x, m1, preferred_element_type=jnp.float32, precision=_HIGH)
    g2 = jnp.dot(x, m2, preferred_element_type=jnp.float32, precision=_HIGH)
    g_ref[...] = jnp.concatenate([g1, g2], axis=1)
    u = jnp.dot(wgat_ref[...], att2_ref[...],
                preferred_element_type=jnp.float32, precision=_HIGH)
    asad_ref[...] = lax.dot_general(
        u, x, (((0,), (1,)), ((), ())),
        preferred_element_type=jnp.float32, precision=_HIGH)


def _prep(xpad, W_gcn, w_gcn, W_gat, w_gat, att2):
    return pl.pallas_call(
        _prep_body,
        out_shape=(
            jax.ShapeDtypeStruct((NPAD, IN_DIM), jnp.float32),
            jax.ShapeDtypeStruct((2, NPAD), jnp.float32),
        ),
    )(xpad, W_gcn, w_gcn, W_gat, w_gat, att2)


@functools.partial(
    pl.kernel,
    out_type=(
        jax.ShapeDtypeStruct((NW, EPW), jnp.float32),
        jax.ShapeDtypeStruct((NC, NPAD), jnp.float32),
        jax.ShapeDtypeStruct((NC, NPAD), jnp.float32),
    ),
    mesh=_mesh,
    scratch_types=(
        pltpu.VMEM((NPAD,), jnp.float32),
        pltpu.VMEM((NPAD,), jnp.float32),
        pltpu.VMEM((NCHUNK, CH), jnp.int32),
        pltpu.VMEM((NCHUNK, CH), jnp.int32),
        pltpu.VMEM((EPW,), jnp.float32),
        pltpu.VMEM((CH,), jnp.float32),
        pltpu.VMEM((NROWS_T,), jnp.float32),
        pltpu.VMEM_SHARED((NPAD,), jnp.float32),
        pltpu.VMEM_SHARED((NPAD,), jnp.float32),
        pltpu.SemaphoreType.DMA,
        pltpu.SemaphoreType.DMA,
    ),
    compiler_params=_sc_params,
)
def _edge_scalar_kernel(srcp, dstp, asad, ee_out, degp, denp,
                        as_v, ad_v, src_v, dst_v, ee_v, ones_v, zero_v,
                        deg_sh, den_sh, sem1, sem2):
    c = lax.axis_index("c")
    s = lax.axis_index("s")
    w = s * NC + c
    zf = jnp.zeros((16,), jnp.float32)
    of = jnp.ones((16,), jnp.float32)

    pltpu.sync_copy(asad.at[0], as_v)
    pltpu.sync_copy(asad.at[1], ad_v)
    pltpu.sync_copy(srcp.at[w], src_v)
    pltpu.sync_copy(dstp.at[w], dst_v)

    @pl.loop(0, CH // 16)
    def _fill_ones(i):
        ones_v[pl.ds(i * 16, 16)] = of

    @pl.loop(0, NROWS_T // 16)
    def _fill_zeros(i):
        zero_v[pl.ds(i * 16, 16)] = zf

    pltpu.sync_copy(zero_v, deg_sh.at[pl.ds(s * NROWS_T, NROWS_T)])
    pltpu.sync_copy(zero_v, den_sh.at[pl.ds(s * NROWS_T, NROWS_T)])
    plsc.subcore_barrier()

    @pl.loop(0, NCHUNK)
    def _chunk(j):
        @pl.loop(0, CH // 16)
        def _grp(g):
            sl = pl.ds(g * 16, 16)
            s16 = src_v[j, sl]
            d16 = dst_v[j, sl]
            asv = plsc.load_gather(as_v, [s16])
            adv = plsc.load_gather(ad_v, [d16])
            e = asv + adv
            e = jnp.maximum(e, 0.2 * e)
            ee = jnp.exp(e)
            ee_v[pl.ds(j * CH + g * 16, 16)] = ee

        cp1 = pltpu.async_copy(ones_v, deg_sh.at[dst_v.at[j]], sem1, add=True)
        cp2 = pltpu.async_copy(ee_v.at[pl.ds(j * CH, CH)],
                               den_sh.at[dst_v.at[j]], sem2, add=True)
        cp1.wait()
        cp2.wait()

    pltpu.sync_copy(ee_v, ee_out.at[w])
    plsc.subcore_barrier()

    pltpu.sync_copy(deg_sh.at[pl.ds(s * NROWS_T, NROWS_T)],
                    degp.at[c, pl.ds(s * NROWS_T, NROWS_T)])
    pltpu.sync_copy(den_sh.at[pl.ds(s * NROWS_T, NROWS_T)],
                    denp.at[c, pl.ds(s * NROWS_T, NROWS_T)])


def _mid_body(degp_ref, denp_ref, asad_ref, g_ref, vec_ref, gp_ref):
    deg = degp_ref[0] + degp_ref[1] + 1.0
    e_self = asad_ref[0] + asad_ref[1]
    e_self = jnp.maximum(e_self, 0.2 * e_self)
    ee_self = jnp.exp(e_self)
    den = denp_ref[0] + denp_ref[1] + ee_self
    dinv = lax.rsqrt(deg)
    rden = 1.0 / den
    invdeg = 1.0 / deg
    selfgat = ee_self * rden
    vec_ref[...] = jnp.stack([dinv, rden, invdeg, selfgat], axis=0)
    g = g_ref[...]
    gp_ref[...] = jnp.concatenate(
        [g[:, :OUT_DIM] * dinv[:, None], g[:, OUT_DIM:]], axis=1)


def _mid(degp, denp, asad, G):
    return pl.pallas_call(
        _mid_body,
        out_shape=(
            jax.ShapeDtypeStruct((4, NPAD), jnp.float32),
            jax.ShapeDtypeStruct((NPAD, IN_DIM), jnp.float32),
        ),
    )(degp, denp, asad, G)


@functools.partial(
    pl.kernel,
    out_type=jax.ShapeDtypeStruct((NC, NPAD, IN_DIM), jnp.float32),
    mesh=_mesh,
    scratch_types=(
        pltpu.VMEM((NCHUNK, CH), jnp.int32),
        pltpu.VMEM((NCHUNK, CH), jnp.int32),
        pltpu.VMEM((EPW,), jnp.float32),
        pltpu.VMEM((CH, IN_DIM), jnp.float32),
        pltpu.VMEM_SHARED((NPAD, IN_DIM), jnp.float32),
        pltpu.SemaphoreType.DMA,
        pltpu.SemaphoreType.DMA,
    ),
    compiler_params=_sc_params,
)
def _edge_row_kernel(srcp, dstp, eep, gp, accp,
                     src_v, dst_v, ee_v, rows, acc_sh, gsem, ssem):
    c = lax.axis_index("c")
    s = lax.axis_index("s")
    w = s * NC + c
    pltpu.sync_copy(srcp.at[w], src_v)
    pltpu.sync_copy(dstp.at[w], dst_v)
    pltpu.sync_copy(eep.at[w], ee_v)

    zf = jnp.zeros((16,), jnp.float32)

    @pl.loop(0, CH)
    def _zero(r):
        for k in range(IN_DIM // 16):
            rows[r, pl.ds(k * 16, 16)] = zf

    for i in range(NROWS_T // CH):
        pltpu.sync_copy(rows, acc_sh.at[pl.ds(s * NROWS_T + i * CH, CH)])
    plsc.subcore_barrier()

    @pl.loop(0, NCHUNK)
    def _chunk(j):
        pltpu.async_copy(gp.at[src_v.at[j]], rows, gsem).wait()

        @pl.loop(0, CH)
        def _scale(e):
            eev = plsc.load_gather(ee_v, [jnp.full((16,), j * CH + e,
                                                   jnp.int32)])
            for k in range(OUT_DIM // 16, IN_DIM // 16):
                sl = pl.ds(k * 16, 16)
                rows[e, sl] = rows[e, sl] * eev

        pltpu.async_copy(rows, acc_sh.at[dst_v.at[j]], ssem, add=True).wait()

    plsc.subcore_barrier()
    pltpu.sync_copy(acc_sh.at[pl.ds(s * NROWS_T, NROWS_T)],
                    accp.at[c, pl.ds(s * NROWS_T, NROWS_T)])


def _final_body(accp_ref, g_ref, vec_ref, b2_ref, wg_ref, wa_ref, out_ref):
    acc = accp_ref[0] + accp_ref[1]
    g = g_ref[...]
    dinv = vec_ref[0]
    rden = vec_ref[1]
    invdeg = vec_ref[2]
    selfgat = vec_ref[3]
    y = (acc[:, :OUT_DIM] * dinv[:, None]
         + acc[:, OUT_DIM:] * rden[:, None]
         + g[:, :OUT_DIM] * invdeg[:, None]
         + g[:, OUT_DIM:] * selfgat[:, None])
    bias = (jnp.dot(b2_ref[0:1], wg_ref[...],
                    preferred_element_type=jnp.float32, precision=_HIGH)
            + jnp.dot(b2_ref[1:2], wa_ref[...],
                      preferred_element_type=jnp.float32, precision=_HIGH))
    out_ref[...] = jnp.tanh(y + bias)[:N]


def _final(accp, G, vec, b2, w_gcn, w_gat):
    return pl.pallas_call(
        _final_body,
        out_shape=jax.ShapeDtypeStruct((N, OUT_DIM), jnp.float32),
    )(accp, G, vec, b2, w_gcn, w_gat)


def kernel(x, edge_index, W_gcn, b_gcn, W_gat, att_src, att_dst, b_gat,
           w_gcn, w_gat):
    xpad = jnp.pad(x, ((0, NPAD - N), (0, 0)))
    pad_idx = jnp.full((EPAD - E,), N, jnp.int32)
    srcp = jnp.concatenate([edge_index[0], pad_idx]).reshape(NW, NCHUNK, CH)
    dstp = jnp.concatenate([edge_index[1], pad_idx]).reshape(NW, NCHUNK, CH)
    att2 = jnp.stack([att_src, att_dst], axis=1)
    b2 = jnp.stack([b_gcn, b_gat], axis=0)

    G, asad = _prep(xpad, W_gcn, w_gcn, W_gat, w_gat, att2)
    ee, degp, denp = _edge_scalar_kernel(srcp, dstp, asad)
    vec, Gp = _mid(degp, denp, asad, G)
    accp = _edge_row_kernel(srcp, dstp, ee.reshape(NW, EPW), Gp)
    return _final(accp, G, vec, b2, w_gcn, w_gat)


# speedup vs baseline: 27.1278x; 1.1817x over previous
import functools

import jax
import jax.numpy as jnp
from jax import lax
from jax.experimental import pallas as pl
from jax.experimental.pallas import tpu as pltpu
from jax.experimental.pallas import tpu_sc as plsc

N = 10000
E = 320000
IN_DIM = 128
OUT_DIM = 64

NC = 2
NS = 16
NW = NC * NS
NPAD = 10240
CH = 64
NCHUNK = 160
EPW = NCHUNK * CH
EPAD = NW * EPW
NROWS_T = NPAD // NS

_HIGH = lax.Precision.HIGHEST

_mesh = plsc.VectorSubcoreMesh(
    core_axis_name="c", subcore_axis_name="s", num_cores=NC, num_subcores=NS
)
_sc_params = pltpu.CompilerParams(
    needs_layout_passes=False, use_tc_tiling_on_sc=False)


def _prep_body(x_ref, wgcn_ref, wg_ref, wgat_ref, wa_ref, att2_ref,
               g_ref, asad_ref):
    x = x_ref[...]
    m1 = jnp.dot(wgcn_ref[...], wg_ref[...],
                 preferred_element_type=jnp.float32, precision=_HIGH)
    m2 = jnp.dot(wgat_ref[...], wa_ref[...],
                 preferred_element_type=jnp.float32, precision=_HIGH)
    g1 = jnp.dot(x, m1, preferred_element_type=jnp.float32, precision=_HIGH)
    g2 = jnp.dot(x, m2, preferred_element_type=jnp.float32, precision=_HIGH)
    g_ref[...] = jnp.concatenate([g1, g2], axis=1)
    u = jnp.dot(wgat_ref[...], att2_ref[...],
                preferred_element_type=jnp.float32, precision=_HIGH)
    asad_ref[...] = lax.dot_general(
        u, x, (((0,), (1,)), ((), ())),
        preferred_element_type=jnp.float32, precision=_HIGH)


def _prep(xpad, W_gcn, w_gcn, W_gat, w_gat, att2):
    return pl.pallas_call(
        _prep_body,
        out_shape=(
            jax.ShapeDtypeStruct((NPAD, IN_DIM), jnp.float32),
            jax.ShapeDtypeStruct((2, NPAD), jnp.float32),
        ),
    )(xpad, W_gcn, w_gcn, W_gat, w_gat, att2)


@functools.partial(
    pl.kernel,
    out_type=(
        jax.ShapeDtypeStruct((NW, EPW), jnp.float32),
        jax.ShapeDtypeStruct((NC, NPAD), jnp.float32),
        jax.ShapeDtypeStruct((NC, NPAD), jnp.float32),
    ),
    mesh=_mesh,
    scratch_types=(
        pltpu.VMEM((NPAD,), jnp.float32),
        pltpu.VMEM((NPAD,), jnp.float32),
        pltpu.VMEM((NCHUNK, CH), jnp.int32),
        pltpu.VMEM((NCHUNK, CH), jnp.int32),
        pltpu.VMEM((EPW,), jnp.float32),
        pltpu.VMEM((CH,), jnp.float32),
        pltpu.VMEM((NROWS_T,), jnp.float32),
        pltpu.VMEM_SHARED((NPAD,), jnp.float32),
        pltpu.VMEM_SHARED((NPAD,), jnp.float32),
        pltpu.SemaphoreType.DMA,
        pltpu.SemaphoreType.DMA,
    ),
    compiler_params=_sc_params,
)
def _edge_scalar_kernel(srcp, dstp, asad, ee_out, degp, denp,
                        as_v, ad_v, src_v, dst_v, ee_v, ones_v, zero_v,
                        deg_sh, den_sh, sem1, sem2):
    c = lax.axis_index("c")
    s = lax.axis_index("s")
    w = s * NC + c
    zf = jnp.zeros((16,), jnp.float32)
    of = jnp.ones((16,), jnp.float32)

    pltpu.sync_copy(asad.at[0], as_v)
    pltpu.sync_copy(asad.at[1], ad_v)
    pltpu.sync_copy(srcp.at[w], src_v)
    pltpu.sync_copy(dstp.at[w], dst_v)

    @pl.loop(0, CH // 16)
    def _fill_ones(i):
        ones_v[pl.ds(i * 16, 16)] = of

    @pl.loop(0, NROWS_T // 16)
    def _fill_zeros(i):
        zero_v[pl.ds(i * 16, 16)] = zf

    pltpu.sync_copy(zero_v, deg_sh.at[pl.ds(s * NROWS_T, NROWS_T)])
    pltpu.sync_copy(zero_v, den_sh.at[pl.ds(s * NROWS_T, NROWS_T)])
    plsc.subcore_barrier()

    @pl.loop(0, NCHUNK)
    def _chunk(j):
        @pl.loop(0, CH // 16)
        def _grp(g):
            sl = pl.ds(g * 16, 16)
            s16 = src_v[j, sl]
            d16 = dst_v[j, sl]
            asv = plsc.load_gather(as_v, [s16])
            adv = plsc.load_gather(ad_v, [d16])
            e = asv + adv
            e = jnp.maximum(e, 0.2 * e)
            ee = jnp.exp(e)
            ee_v[pl.ds(j * CH + g * 16, 16)] = ee

        cp1 = pltpu.async_copy(ones_v, deg_sh.at[dst_v.at[j]], sem1, add=True)
        cp2 = pltpu.async_copy(ee_v.at[pl.ds(j * CH, CH)],
                               den_sh.at[dst_v.at[j]], sem2, add=True)
        cp1.wait()
        cp2.wait()

    pltpu.sync_copy(ee_v, ee_out.at[w])
    plsc.subcore_barrier()

    pltpu.sync_copy(deg_sh.at[pl.ds(s * NROWS_T, NROWS_T)],
                    degp.at[c, pl.ds(s * NROWS_T, NROWS_T)])
    pltpu.sync_copy(den_sh.at[pl.ds(s * NROWS_T, NROWS_T)],
                    denp.at[c, pl.ds(s * NROWS_T, NROWS_T)])


def _mid_body(degp_ref, denp_ref, asad_ref, g_ref, vec_ref, gp_ref):
    deg = degp_ref[0] + degp_ref[1] + 1.0
    e_self = asad_ref[0] + asad_ref[1]
    e_self = jnp.maximum(e_self, 0.2 * e_self)
    ee_self = jnp.exp(e_self)
    den = denp_ref[0] + denp_ref[1] + ee_self
    dinv = lax.rsqrt(deg)
    rden = 1.0 / den
    invdeg = 1.0 / deg
    selfgat = ee_self * rden
    vec_ref[...] = jnp.stack([dinv, rden, invdeg, selfgat], axis=0)
    g = g_ref[...]
    gp_ref[...] = jnp.concatenate(
        [g[:, :OUT_DIM] * dinv[:, None], g[:, OUT_DIM:]], axis=1)


def _mid(degp, denp, asad, G):
    return pl.pallas_call(
        _mid_body,
        out_shape=(
            jax.ShapeDtypeStruct((4, NPAD), jnp.float32),
            jax.ShapeDtypeStruct((NPAD, IN_DIM), jnp.float32),
        ),
    )(degp, denp, asad, G)


@functools.partial(
    pl.kernel,
    out_type=jax.ShapeDtypeStruct((NC, NPAD, IN_DIM), jnp.float32),
    mesh=_mesh,
    scratch_types=(
        pltpu.VMEM((NCHUNK, CH), jnp.int32),
        pltpu.VMEM((NCHUNK, CH), jnp.int32),
        pltpu.VMEM((EPW,), jnp.float32),
        pltpu.VMEM((CH, IN_DIM), jnp.float32),
        pltpu.VMEM((CH, IN_DIM), jnp.float32),
        pltpu.VMEM_SHARED((NPAD, IN_DIM), jnp.float32),
        pltpu.SemaphoreType.DMA,
        pltpu.SemaphoreType.DMA,
        pltpu.SemaphoreType.DMA,
        pltpu.SemaphoreType.DMA,
    ),
    compiler_params=_sc_params,
)
def _edge_row_kernel(srcp, dstp, eep, gp, accp,
                     src_v, dst_v, ee_v, rows0, rows1, acc_sh,
                     gs0, gs1, ss0, ss1):
    c = lax.axis_index("c")
    s = lax.axis_index("s")
    w = s * NC + c
    pltpu.sync_copy(srcp.at[w], src_v)
    pltpu.sync_copy(dstp.at[w], dst_v)
    pltpu.sync_copy(eep.at[w], ee_v)

    zf = jnp.zeros((16,), jnp.float32)

    pltpu.async_copy(gp.at[src_v.at[0]], rows0, gs0)

    @pl.loop(0, CH)
    def _zero(r):
        for k in range(IN_DIM // 16):
            rows1[r, pl.ds(k * 16, 16)] = zf

    for i in range(NROWS_T // CH):
        pltpu.sync_copy(rows1, acc_sh.at[pl.ds(s * NROWS_T + i * CH, CH)])
    plsc.subcore_barrier()

    bufs = (rows0, rows1)
    gsems = (gs0, gs1)
    ssems = (ss0, ss1)

    @pl.loop(0, NCHUNK // 2)
    def _pair(t):
        for b in range(2):
            jj = t * 2 + b
            rb, gb, sb = bufs[b], gsems[b], ssems[b]
            ro, go, so = bufs[1 - b], gsems[1 - b], ssems[1 - b]

            pltpu.make_async_copy(gp.at[src_v.at[jj]], rb, gb).wait()

            @pl.when(jnp.logical_and(jj >= 1, jj + 1 < NCHUNK))
            def _drain():
                pltpu.make_async_copy(
                    ro, acc_sh.at[dst_v.at[jj - 1]], so).wait()

            @pl.when(jj + 1 < NCHUNK)
            def _prefetch():
                pltpu.async_copy(gp.at[src_v.at[jj + 1]], ro, go)

            @pl.loop(0, CH, unroll=4)
            def _scale(e):
                eev = plsc.load_gather(
                    ee_v, [jnp.full((16,), jj * CH + e, jnp.int32)])
                for k in range(OUT_DIM // 16, IN_DIM // 16):
                    sl = pl.ds(k * 16, 16)
                    rb[e, sl] = rb[e, sl] * eev

            pltpu.async_copy(rb, acc_sh.at[dst_v.at[jj]], sb, add=True)

    pltpu.make_async_copy(rows0, acc_sh.at[dst_v.at[NCHUNK - 2]], ss0).wait()
    pltpu.make_async_copy(rows1, acc_sh.at[dst_v.at[NCHUNK - 1]], ss1).wait()
    plsc.subcore_barrier()
    pltpu.sync_copy(acc_sh.at[pl.ds(s * NROWS_T, NROWS_T)],
                    accp.at[c, pl.ds(s * NROWS_T, NROWS_T)])


def _final_body(accp_ref, g_ref, vec_ref, b2_ref, wg_ref, wa_ref, out_ref):
    acc = accp_ref[0] + accp_ref[1]
    g = g_ref[...]
    dinv = vec_ref[0]
    rden = vec_ref[1]
    invdeg = vec_ref[2]
    selfgat = vec_ref[3]
    y = (acc[:, :OUT_DIM] * dinv[:, None]
         + acc[:, OUT_DIM:] * rden[:, None]
         + g[:, :OUT_DIM] * invdeg[:, None]
         + g[:, OUT_DIM:] * selfgat[:, None])
    bias = (jnp.dot(b2_ref[0:1], wg_ref[...],
                    preferred_element_type=jnp.float32, precision=_HIGH)
            + jnp.dot(b2_ref[1:2], wa_ref[...],
                      preferred_element_type=jnp.float32, precision=_HIGH))
    out_ref[...] = jnp.tanh(y + bias)[:N]


def _final(accp, G, vec, b2, w_gcn, w_gat):
    return pl.pallas_call(
        _final_body,
        out_shape=jax.ShapeDtypeStruct((N, OUT_DIM), jnp.float32),
    )(accp, G, vec, b2, w_gcn, w_gat)


def kernel(x, edge_index, W_gcn, b_gcn, W_gat, att_src, att_dst, b_gat,
           w_gcn, w_gat):
    xpad = jnp.pad(x, ((0, NPAD - N), (0, 0)))
    pad_idx = jnp.full((EPAD - E,), N, jnp.int32)
    srcp = jnp.concatenate([edge_index[0], pad_idx]).reshape(NW, NCHUNK, CH)
    dstp = jnp.concatenate([edge_index[1], pad_idx]).reshape(NW, NCHUNK, CH)
    att2 = jnp.stack([att_src, att_dst], axis=1)
    b2 = jnp.stack([b_gcn, b_gat], axis=0)

    G, asad = _prep(xpad, W_gcn, w_gcn, W_gat, w_gat, att2)
    ee, degp, denp = _edge_scalar_kernel(srcp, dstp, asad)
    vec, Gp = _mid(degp, denp, asad, G)
    accp = _edge_row_kernel(srcp, dstp, ee.reshape(NW, EPW), Gp)
    return _final(accp, G, vec, b2, w_gcn, w_gat)


# speedup vs baseline: 34.8545x; 1.2848x over previous
import functools

import numpy as np

import jax
import jax.numpy as jnp
from jax import lax
from jax.experimental import pallas as pl
from jax.experimental.pallas import tpu as pltpu
from jax.experimental.pallas import tpu_sc as plsc

N = 10000
E = 320000
IN_DIM = 128
OUT_DIM = 64

NC = 2
NS = 16
NW = NC * NS
NPAD = 10240
CH = 64
NCHUNK = 160
EPW = NCHUNK * CH
EPAD = NW * EPW
NROWS_T = NPAD // NS

_HIGH = lax.Precision.HIGHEST

_mesh = plsc.VectorSubcoreMesh(
    core_axis_name="c", subcore_axis_name="s", num_cores=NC, num_subcores=NS
)
_sc_params = pltpu.CompilerParams(
    needs_layout_passes=False, use_tc_tiling_on_sc=False)

_PERM = np.empty(IN_DIM, np.int64)
for _k in range(IN_DIM // 32):
    for _i in range(16):
        _PERM[32 * _k + 2 * _i] = 32 * _k + _i
        _PERM[32 * _k + 2 * _i + 1] = 32 * _k + 16 + _i
_PMAT = np.zeros((IN_DIM, IN_DIM), np.float32)
_PMAT[_PERM, np.arange(IN_DIM)] = 1.0


def _prep_body(x_ref, wgcn_ref, wg_ref, wgat_ref, wa_ref, att2_ref,
               g_ref, asad_ref):
    x = x_ref[...]
    m1 = jnp.dot(wgcn_ref[...], wg_ref[...],
                 preferred_element_type=jnp.float32, precision=_HIGH)
    m2 = jnp.dot(wgat_ref[...], wa_ref[...],
                 preferred_element_type=jnp.float32, precision=_HIGH)
    g1 = jnp.dot(x, m1, preferred_element_type=jnp.float32, precision=_HIGH)
    g2 = jnp.dot(x, m2, preferred_element_type=jnp.float32, precision=_HIGH)
    g_ref[...] = jnp.concatenate([g1, g2], axis=1)
    u = jnp.dot(wgat_ref[...], att2_ref[...],
                preferred_element_type=jnp.float32, precision=_HIGH)
    asad_ref[...] = lax.dot_general(
        u, x, (((0,), (1,)), ((), ())),
        preferred_element_type=jnp.float32, precision=_HIGH)


def _prep(xpad, W_gcn, w_gcn, W_gat, w_gat, att2):
    return pl.pallas_call(
        _prep_body,
        out_shape=(
            jax.ShapeDtypeStruct((NPAD, IN_DIM), jnp.float32),
            jax.ShapeDtypeStruct((2, NPAD), jnp.float32),
        ),
    )(xpad, W_gcn, w_gcn, W_gat, w_gat, att2)


@functools.partial(
    pl.kernel,
    out_type=(
        jax.ShapeDtypeStruct((NW, EPW), jnp.float32),
        jax.ShapeDtypeStruct((NC, NPAD), jnp.float32),
        jax.ShapeDtypeStruct((NC, NPAD), jnp.float32),
    ),
    mesh=_mesh,
    scratch_types=(
        pltpu.VMEM((NPAD,), jnp.float32),
        pltpu.VMEM((NPAD,), jnp.float32),
        pltpu.VMEM((NCHUNK, CH), jnp.int32),
        pltpu.VMEM((NCHUNK, CH), jnp.int32),
        pltpu.VMEM((EPW,), jnp.float32),
        pltpu.VMEM((CH,), jnp.float32),
        pltpu.VMEM((NROWS_T,), jnp.float32),
        pltpu.VMEM_SHARED((NPAD,), jnp.float32),
        pltpu.VMEM_SHARED((NPAD,), jnp.float32),
        pltpu.SemaphoreType.DMA,
        pltpu.SemaphoreType.DMA,
    ),
    compiler_params=_sc_params,
)
def _edge_scalar_kernel(srcp, dstp, asad, ee_out, degp, denp,
                        as_v, ad_v, src_v, dst_v, ee_v, ones_v, zero_v,
                        deg_sh, den_sh, sem1, sem2):
    c = lax.axis_index("c")
    s = lax.axis_index("s")
    w = s * NC + c
    zf = jnp.zeros((16,), jnp.float32)
    of = jnp.ones((16,), jnp.float32)

    pltpu.sync_copy(asad.at[0], as_v)
    pltpu.sync_copy(asad.at[1], ad_v)
    pltpu.sync_copy(srcp.at[w], src_v)
    pltpu.sync_copy(dstp.at[w], dst_v)

    @pl.loop(0, CH // 16)
    def _fill_ones(i):
        ones_v[pl.ds(i * 16, 16)] = of

    @pl.loop(0, NROWS_T // 16)
    def _fill_zeros(i):
        zero_v[pl.ds(i * 16, 16)] = zf

    pltpu.sync_copy(zero_v, deg_sh.at[pl.ds(s * NROWS_T, NROWS_T)])
    pltpu.sync_copy(zero_v, den_sh.at[pl.ds(s * NROWS_T, NROWS_T)])
    plsc.subcore_barrier()

    @pl.loop(0, NCHUNK)
    def _chunk(j):
        @pl.loop(0, CH // 16)
        def _grp(g):
            sl = pl.ds(g * 16, 16)
            s16 = src_v[j, sl]
            d16 = dst_v[j, sl]
            asv = plsc.load_gather(as_v, [s16])
            adv = plsc.load_gather(ad_v, [d16])
            e = asv + adv
            e = jnp.maximum(e, 0.2 * e)
            ee = jnp.exp(e)
            ee_v[pl.ds(j * CH + g * 16, 16)] = ee

        cp1 = pltpu.async_copy(ones_v, deg_sh.at[dst_v.at[j]], sem1, add=True)
        cp2 = pltpu.async_copy(ee_v.at[pl.ds(j * CH, CH)],
                               den_sh.at[dst_v.at[j]], sem2, add=True)
        cp1.wait()
        cp2.wait()

    pltpu.sync_copy(ee_v, ee_out.at[w])
    plsc.subcore_barrier()

    pltpu.sync_copy(deg_sh.at[pl.ds(s * NROWS_T, NROWS_T)],
                    degp.at[c, pl.ds(s * NROWS_T, NROWS_T)])
    pltpu.sync_copy(den_sh.at[pl.ds(s * NROWS_T, NROWS_T)],
                    denp.at[c, pl.ds(s * NROWS_T, NROWS_T)])


def _mid_body(degp_ref, denp_ref, asad_ref, g_ref, pmat_ref, vec_ref, gp_ref):
    deg = degp_ref[0] + degp_ref[1] + 1.0
    e_self = asad_ref[0] + asad_ref[1]
    e_self = jnp.maximum(e_self, 0.2 * e_self)
    ee_self = jnp.exp(e_self)
    den = denp_ref[0] + denp_ref[1] + ee_self
    dinv = lax.rsqrt(deg)
    rden = 1.0 / den
    invdeg = 1.0 / deg
    selfgat = ee_self * rden
    vec_ref[...] = jnp.stack([dinv, rden, invdeg, selfgat], axis=0)
    g = g_ref[...]
    gp = jnp.concatenate(
        [g[:, :OUT_DIM] * dinv[:, None], g[:, OUT_DIM:]], axis=1)
    gp_ref[...] = jnp.dot(gp, pmat_ref[...],
                          preferred_element_type=jnp.float32,
                          precision=_HIGH).astype(jnp.bfloat16)


def _mid(degp, denp, asad, G, pmat):
    return pl.pallas_call(
        _mid_body,
        out_shape=(
            jax.ShapeDtypeStruct((4, NPAD), jnp.float32),
            jax.ShapeDtypeStruct((NPAD, IN_DIM), jnp.bfloat16),
        ),
    )(degp, denp, asad, G, pmat)


@functools.partial(
    pl.kernel,
    out_type=jax.ShapeDtypeStruct((NC, NPAD, IN_DIM), jnp.float32),
    mesh=_mesh,
    scratch_types=(
        pltpu.VMEM((NCHUNK, CH), jnp.int32),
        pltpu.VMEM((NCHUNK, CH), jnp.int32),
        pltpu.VMEM((CH,), jnp.float32),
        pltpu.VMEM((CH,), jnp.float32),
        pltpu.VMEM((CH, IN_DIM), jnp.bfloat16),
        pltpu.VMEM((CH, IN_DIM), jnp.bfloat16),
        pltpu.VMEM((CH, IN_DIM), jnp.float32),
        pltpu.VMEM((CH, IN_DIM), jnp.float32),
        pltpu.VMEM_SHARED((NPAD, IN_DIM), jnp.float32),
        pltpu.SemaphoreType.DMA,
        pltpu.SemaphoreType.DMA,
        pltpu.SemaphoreType.DMA,
        pltpu.SemaphoreType.DMA,
    ),
    compiler_params=_sc_params,
)
def _edge_row_kernel(srcp, dstp, eep, gp, accp,
                     src_v, dst_v, ee0, ee1, rbf0, rbf1, rf0, rf1, acc_sh,
                     gs0, gs1, ss0, ss1):
    c = lax.axis_index("c")
    s = lax.axis_index("s")
    w = s * NC + c
    pltpu.sync_copy(srcp.at[w], src_v)
    pltpu.sync_copy(dstp.at[w], dst_v)

    zf = jnp.zeros((16,), jnp.float32)

    pltpu.async_copy(gp.at[src_v.at[0]], rbf0, gs0)
    pltpu.async_copy(eep.at[w, pl.ds(0, CH)], ee0, gs0)

    @pl.loop(0, CH)
    def _zero(r):
        for k in range(IN_DIM // 16):
            rf1[r, pl.ds(k * 16, 16)] = zf

    for i in range(NROWS_T // CH):
        pltpu.sync_copy(rf1, acc_sh.at[pl.ds(s * NROWS_T + i * CH, CH)])
    plsc.subcore_barrier()

    bf_bufs = (rbf0, rbf1)
    f_bufs = (rf0, rf1)
    ee_bufs = (ee0, ee1)
    gsems = (gs0, gs1)
    ssems = (ss0, ss1)

    @pl.loop(0, NCHUNK // 2)
    def _pair(t):
        for b in range(2):
            jj = t * 2 + b
            rbf, rf, eb = bf_bufs[b], f_bufs[b], ee_bufs[b]
            gb, sb = gsems[b], ssems[b]
            rbf_o, rf_o, eb_o = bf_bufs[1 - b], f_bufs[1 - b], ee_bufs[1 - b]
            go, so = gsems[1 - b], ssems[1 - b]

            pltpu.make_async_copy(gp.at[src_v.at[jj]], rbf, gb).wait()
            pltpu.make_async_copy(eep.at[w, pl.ds(jj * CH, CH)], eb,
                                  gb).wait()

            @pl.when(jnp.logical_and(jj >= 1, jj + 1 < NCHUNK))
            def _drain():
                pltpu.make_async_copy(
                    rf_o, acc_sh.at[dst_v.at[jj - 1]], so).wait()

            @pl.when(jj + 1 < NCHUNK)
            def _prefetch():
                pltpu.async_copy(gp.at[src_v.at[jj + 1]], rbf_o, go)
                pltpu.async_copy(eep.at[w, pl.ds((jj + 1) * CH, CH)], eb_o,
                                 go)

            @pl.loop(0, CH, unroll=2)
            def _scale(e):
                eev = plsc.load_gather(eb, [jnp.full((16,), e, jnp.int32)])
                for k in range(IN_DIM // 32):
                    v = rbf[e, pl.ds(k * 32, 32)]
                    va, vb = plsc.unpack(
                        v, format=plsc.PackFormat.INTERLEAVED)
                    if k >= OUT_DIM // 32:
                        va = va * eev
                        vb = vb * eev
                    rf[e, pl.ds(k * 32, 16)] = va
                    rf[e, pl.ds(k * 32 + 16, 16)] = vb

            pltpu.async_copy(rf, acc_sh.at[dst_v.at[jj]], sb, add=True)

    pltpu.make_async_copy(rf0, acc_sh.at[dst_v.at[NCHUNK - 2]], ss0).wait()
    pltpu.make_async_copy(rf1, acc_sh.at[dst_v.at[NCHUNK - 1]], ss1).wait()
    plsc.subcore_barrier()
    pltpu.sync_copy(acc_sh.at[pl.ds(s * NROWS_T, NROWS_T)],
                    accp.at[c, pl.ds(s * NROWS_T, NROWS_T)])


def _final_body(accp_ref, g_ref, vec_ref, b2_ref, wg_ref, wa_ref, out_ref):
    acc = accp_ref[0] + accp_ref[1]
    g = g_ref[...]
    dinv = vec_ref[0]
    rden = vec_ref[1]
    invdeg = vec_ref[2]
    selfgat = vec_ref[3]
    y = (acc[:, :OUT_DIM] * dinv[:, None]
         + acc[:, OUT_DIM:] * rden[:, None]
         + g[:, :OUT_DIM] * invdeg[:, None]
         + g[:, OUT_DIM:] * selfgat[:, None])
    bias = (jnp.dot(b2_ref[0:1], wg_ref[...],
                    preferred_element_type=jnp.float32, precision=_HIGH)
            + jnp.dot(b2_ref[1:2], wa_ref[...],
                      preferred_element_type=jnp.float32, precision=_HIGH))
    out_ref[...] = jnp.tanh(y + bias)[:N]


def _final(accp, G, vec, b2, w_gcn, w_gat):
    return pl.pallas_call(
        _final_body,
        out_shape=jax.ShapeDtypeStruct((N, OUT_DIM), jnp.float32),
    )(accp, G, vec, b2, w_gcn, w_gat)


def kernel(x, edge_index, W_gcn, b_gcn, W_gat, att_src, att_dst, b_gat,
           w_gcn, w_gat):
    xpad = jnp.pad(x, ((0, NPAD - N), (0, 0)))
    pad_idx = jnp.full((EPAD - E,), N, jnp.int32)
    srcp = jnp.concatenate([edge_index[0], pad_idx]).reshape(NW, NCHUNK, CH)
    dstp = jnp.concatenate([edge_index[1], pad_idx]).reshape(NW, NCHUNK, CH)
    att2 = jnp.stack([att_src, att_dst], axis=1)
    b2 = jnp.stack([b_gcn, b_gat], axis=0)

    G, asad = _prep(xpad, W_gcn, w_gcn, W_gat, w_gat, att2)
    ee, degp, denp = _edge_scalar_kernel(srcp, dstp, asad)
    vec, Gp = _mid(degp, denp, asad, G, jnp.asarray(_PMAT))
    accp = _edge_row_kernel(srcp, dstp, ee, Gp)
    return _final(accp, G, vec, b2, w_gcn, w_gat)
